# Initial kernel scaffold; baseline (speedup 1.0000x reference)
import jax
import jax.numpy as jnp
from jax.experimental import pallas as pl


def kernel(xyz, points, params):
    raise NotImplementedError("write your pallas kernel here")


# speedup vs baseline: 6.3197x; 6.3197x over previous
import functools

import jax
import jax.numpy as jnp
from jax.experimental import pallas as pl
from jax.experimental.pallas import tpu as pltpu

_BN_EPS = 1e-5
_F32 = jnp.float32
_I32 = jnp.int32


def _fps_body(xyz_ref, out_ref, *, npoint, n, b):
    n8 = n // 8
    x = xyz_ref[:, 0]
    y = xyz_ref[:, 1]
    z = xyz_ref[:, 2]
    row = jax.lax.broadcasted_iota(_I32, (b, 8, n8), 1)
    col = jax.lax.broadcasted_iota(_I32, (b, 8, n8), 2)
    lin = row * n8 + col

    def body(i, carry):
        dmin, far = carry
        for bb in range(b):
            out_ref[pl.ds(i, 1), bb:bb + 1] = far[bb]
        sel = lin == far
        cx = jnp.sum(jnp.where(sel, x, 0.0), axis=(1, 2), keepdims=True)
        cy = jnp.sum(jnp.where(sel, y, 0.0), axis=(1, 2), keepdims=True)
        cz = jnp.sum(jnp.where(sel, z, 0.0), axis=(1, 2), keepdims=True)
        dx = x - cx
        dy = y - cy
        dz = z - cz
        dist = dx * dx + dy * dy + dz * dz
        dmin = jnp.minimum(dmin, dist)
        m = jnp.max(dmin, axis=(1, 2), keepdims=True)
        far = jnp.min(jnp.where(dmin == m, lin, n), axis=(1, 2),
                      keepdims=True).astype(_I32)
        return dmin, far

    init = (jnp.full((b, 8, n8), 1e10, _F32), jnp.zeros((b, 1, 1), _I32))
    jax.lax.fori_loop(0, npoint, body, init)


def _fps(xyz_cn, npoint):
    b, _, n = xyz_cn.shape
    xyz_r = xyz_cn.reshape(b, 3, 8, n // 8)
    out = pl.pallas_call(
        functools.partial(_fps_body, npoint=npoint, n=n, b=b),
        grid=(1,),
        in_specs=[pl.BlockSpec((b, 3, 8, n // 8), lambda i: (0, 0, 0, 0))],
        out_specs=pl.BlockSpec((npoint, b), lambda i: (0, 0)),
        out_shape=jax.ShapeDtypeStruct((npoint, b), _I32),
    )(xyz_r)
    return out.T


def _cumsum_lanes(x):
    n = x.shape[1]
    sh = 1
    while sh < n:
        pad = jnp.zeros((x.shape[0], sh), x.dtype)
        x = x + jnp.concatenate([pad, x[:, :-sh]], axis=1)
        sh *= 2
    return x


def _bq_body(nx_ref, xyz_ref, out_ref, *, n, k, r2):
    c = nx_ref[0]
    p = xyz_ref[0]
    cn = jnp.sum(c * c, axis=1, keepdims=True)
    pn = jnp.sum(p * p, axis=0, keepdims=True)
    mm = jax.lax.dot_general(c, p, (((1,), (0,)), ((), ())),
                             preferred_element_type=_F32)
    d = cn + pn - 2.0 * mm
    mask = (d <= r2).astype(_I32)
    e = _cumsum_lanes(mask)
    first = None
    for kk in range(k):
        cnt = jnp.sum((e <= kk).astype(_I32), axis=1, keepdims=True)
        if kk == 0:
            first = cnt
            colv = cnt
        else:
            colv = jnp.where(cnt >= n, first, cnt)
        out_ref[0, :, kk:kk + 1] = colv


def _bq(new_xyz_rows, xyz_cn, radius, k, s_t):
    b, s, _ = new_xyz_rows.shape
    n = xyz_cn.shape[2]
    grid = (b, s // s_t)
    return pl.pallas_call(
        functools.partial(_bq_body, n=n, k=k, r2=float(radius) ** 2),
        grid=grid,
        in_specs=[
            pl.BlockSpec((1, s_t, 3), lambda bb, ss: (bb, ss, 0)),
            pl.BlockSpec((1, 3, n), lambda bb, ss: (bb, 0, 0)),
        ],
        out_specs=pl.BlockSpec((1, s_t, k), lambda bb, ss: (bb, ss, 0)),
        out_shape=jax.ShapeDtypeStruct((b, s, k), _I32),
    )(new_xyz_rows, xyz_cn)


def _gather_body(*refs, nc, nchunks, has_sub):
    if has_sub:
        idx_ref, feat_ref, sub_ref, out_ref = refs
    else:
        idx_ref, feat_ref, out_ref = refs
        sub_ref = None
    j = pl.program_id(2)

    @pl.when(j == 0)
    def _():
        out_ref[...] = jnp.zeros_like(out_ref)

    idx = idx_ref[0]
    base = j * nc
    ii = jax.lax.broadcasted_iota(_I32, (1, nc), 1) + base
    oh = (idx == ii).astype(_F32)
    out_ref[0] += jax.lax.dot_general(
        oh, feat_ref[0], (((1,), (0,)), ((), ())),
        preferred_element_type=_F32,
        precision=jax.lax.Precision.HIGHEST)
    if has_sub:
        @pl.when(j == nchunks - 1)
        def _():
            out_ref[0] -= sub_ref[0]


def _gather(feat_rows, idx, sub=None):
    b, n, c = feat_rows.shape
    m = idx.shape[1]
    m_t = min(m, 1024)
    nc = min(n, 2048)
    idx3 = idx.reshape(b, m, 1)
    grid = (b, m // m_t, n // nc)
    in_specs = [
        pl.BlockSpec((1, m_t, 1), lambda bb, mm, jj: (bb, mm, 0)),
        pl.BlockSpec((1, nc, c), lambda bb, mm, jj: (bb, jj, 0)),
    ]
    args = [idx3, feat_rows]
    if sub is not None:
        in_specs.append(pl.BlockSpec((1, m_t, c), lambda bb, mm, jj: (bb, mm, 0)))
        args.append(sub)
    return pl.pallas_call(
        functools.partial(_gather_body, nc=nc, nchunks=n // nc,
                          has_sub=sub is not None),
        grid=grid,
        in_specs=in_specs,
        out_specs=pl.BlockSpec((1, m_t, c), lambda bb, mm, jj: (bb, mm, 0)),
        out_shape=jax.ShapeDtypeStruct((b, m, c), _F32),
    )(*args)


def _layer_body(*refs, nin, norm_flags):
    refs = list(refs)
    y_ref, sum_ref = refs[-2:]
    pos = 0
    acc = None
    for i in range(nin):
        x_ref = refs[pos]
        w_ref = refs[pos + 1]
        pos += 2
        v = x_ref[...]
        if norm_flags[i]:
            mu_ref = refs[pos]
            inv_ref = refs[pos + 1]
            pos += 2
            v = jnp.maximum((v - mu_ref[...]) / inv_ref[...], 0.0)
        t = jax.lax.dot_general(v, w_ref[...], (((1,), (0,)), ((), ())),
                                preferred_element_type=_F32)
        acc = t if acc is None else acc + t
    y_ref[...] = acc

    @pl.when(pl.program_id(0) == 0)
    def _():
        sum_ref[...] = jnp.zeros_like(sum_ref)

    sum_ref[...] += jnp.sum(acc, axis=0, keepdims=True)


def _layer(xs, wts, norms, r_t=2048):
    rt = xs[0].shape[0]
    o = wts[0].shape[1]
    r_t = min(rt, r_t)
    grid = (rt // r_t,)
    in_specs = []
    args = []
    norm_flags = []
    for x, wt, nrm in zip(xs, wts, norms):
        ci = x.shape[1]
        in_specs.append(pl.BlockSpec((r_t, ci), lambda i: (i, 0)))
        in_specs.append(pl.BlockSpec((ci, o), lambda i: (0, 0)))
        args += [x, wt]
        norm_flags.append(nrm is not None)
        if nrm is not None:
            in_specs.append(pl.BlockSpec((1, ci), lambda i: (0, 0)))
            in_specs.append(pl.BlockSpec((1, ci), lambda i: (0, 0)))
            args += [nrm[0], nrm[1]]
    out_specs = (
        pl.BlockSpec((r_t, o), lambda i: (i, 0)),
        pl.BlockSpec((1, o), lambda i: (0, 0)),
    )
    out_shape = (
        jax.ShapeDtypeStruct((rt, o), _F32),
        jax.ShapeDtypeStruct((1, o), _F32),
    )
    return pl.pallas_call(
        functools.partial(_layer_body, nin=len(xs),
                          norm_flags=tuple(norm_flags)),
        grid=grid,
        in_specs=in_specs,
        out_specs=out_specs,
        out_shape=out_shape,
    )(*args)


def _var_body(x_ref, mu_ref, ss_ref):
    @pl.when(pl.program_id(0) == 0)
    def _():
        ss_ref[...] = jnp.zeros_like(ss_ref)

    d = x_ref[...] - mu_ref[...]
    ss_ref[...] += jnp.sum(d * d, axis=0, keepdims=True)


def _varpass(x, mu, r_t=2048):
    rt, o = x.shape
    r_t = min(rt, r_t)
    return pl.pallas_call(
        _var_body,
        grid=(rt // r_t,),
        in_specs=[
            pl.BlockSpec((r_t, o), lambda i: (i, 0)),
            pl.BlockSpec((1, o), lambda i: (0, 0)),
        ],
        out_specs=pl.BlockSpec((1, o), lambda i: (0, 0)),
        out_shape=jax.ShapeDtypeStruct((1, o), _F32),
    )(x, mu)


def _stats(y, s, cnt):
    mu = s / cnt
    ssd = _varpass(y, mu)
    sig = jnp.sqrt(ssd / cnt + _BN_EPS)
    return mu, sig


def _final_body(x_ref, mu_ref, inv_ref, out_ref, *, k):
    v = jnp.maximum((x_ref[...] - mu_ref[...]) / inv_ref[...], 0.0)
    if k:
        out_ref[...] = jnp.max(v.reshape(-1, k, v.shape[-1]), axis=1)
    else:
        out_ref[...] = v


def _final(x, mu, inv, k=None, r_t=2048):
    rt, o = x.shape
    r_t = min(rt, r_t)
    grid = (rt // r_t,)
    if k:
        out_specs = pl.BlockSpec((r_t // k, o), lambda i: (i, 0))
        out_shape = jax.ShapeDtypeStruct((rt // k, o), _F32)
    else:
        out_specs = pl.BlockSpec((r_t, o), lambda i: (i, 0))
        out_shape = jax.ShapeDtypeStruct((rt, o), _F32)
    return pl.pallas_call(
        functools.partial(_final_body, k=k),
        grid=grid,
        in_specs=[
            pl.BlockSpec((r_t, o), lambda i: (i, 0)),
            pl.BlockSpec((1, o), lambda i: (0, 0)),
            pl.BlockSpec((1, o), lambda i: (0, 0)),
        ],
        out_specs=out_specs,
        out_shape=out_shape,
    )(x, mu, inv)


def _interp_body(x1_ref, x2_ref, p2_ref, out_ref, *, s2):
    c = x1_ref[0]
    p = x2_ref[0]
    f = p2_ref[0]
    cn = jnp.sum(c * c, axis=1, keepdims=True)
    pn = jnp.sum(p * p, axis=0, keepdims=True)
    mm = jax.lax.dot_general(c, p, (((1,), (0,)), ((), ())),
                             preferred_element_type=_F32)
    d = cn + pn - 2.0 * mm
    cols = jax.lax.broadcasted_iota(_I32, d.shape, 1)
    dd = d
    recips = []
    onehots = []
    for _ in range(3):
        m = jnp.min(dd, axis=1, keepdims=True)
        idx = jnp.min(jnp.where(dd == m, cols, s2), axis=1, keepdims=True)
        oh = cols == idx
        recips.append(1.0 / (m + 1e-8))
        onehots.append(oh)
        dd = jnp.where(oh, jnp.float32(1e30), dd)
    rsum = recips[0] + recips[1] + recips[2]
    acc = None
    for t in range(3):
        g = jax.lax.dot_general(onehots[t].astype(_F32), f,
                                (((1,), (0,)), ((), ())),
                                preferred_element_type=_F32,
                                precision=jax.lax.Precision.HIGHEST)
        term = g * (recips[t] / rsum)
        acc = term if acc is None else acc + term
    out_ref[0] = acc


def _interp(x1_rows, x2_cn, p2_rows, n_t):
    b, n1, _ = x1_rows.shape
    s2 = x2_cn.shape[2]
    c2 = p2_rows.shape[2]
    n_t = min(n1, n_t)
    grid = (b, n1 // n_t)
    return pl.pallas_call(
        functools.partial(_interp_body, s2=s2),
        grid=grid,
        in_specs=[
            pl.BlockSpec((1, n_t, 3), lambda bb, ii: (bb, ii, 0)),
            pl.BlockSpec((1, 3, s2), lambda bb, ii: (bb, 0, 0)),
            pl.BlockSpec((1, s2, c2), lambda bb, ii: (bb, 0, 0)),
        ],
        out_specs=pl.BlockSpec((1, n_t, c2), lambda bb, ii: (bb, ii, 0)),
        out_shape=jax.ShapeDtypeStruct((b, n1, c2), _F32),
    )(x1_rows, x2_cn, p2_rows)


def _sa_stage(xyz_cn, xyz_rows, feat_rows, npoint, radius, k, layers, s_t):
    b, n, cin = feat_rows.shape
    fps_idx = _fps(xyz_cn, npoint)
    new_xyz = _gather(xyz_rows, fps_idx)
    idx = _bq(new_xyz, xyz_cn, radius, k, s_t)
    sub = jnp.pad(jnp.repeat(new_xyz, k, axis=1), ((0, 0), (0, 0), (0, cin - 3)))
    grouped = _gather(feat_rows, idx.reshape(b, npoint * k), sub)
    rt = b * npoint * k
    g = grouped.reshape(rt, cin)

    y, s = _layer([g], [layers[0]['W'].T], [None])
    mu, sig = _stats(y, s, rt)
    for lp in layers[1:]:
        y, s = _layer([y], [lp['W'].T], [(mu, sig)])
        mu, sig = _stats(y, s, rt)
    o = y.shape[1]
    pooled = _final(y, mu, sig, k=k)
    return new_xyz, pooled.reshape(b, npoint, o)


def _fp_stage(x1_rows, x2_cn, p1_rows, p2_rows, layers, n_t):
    b, n1, _ = x1_rows.shape
    interp = _interp(x1_rows, x2_cn, p2_rows, n_t)
    rt = b * n1
    if p1_rows is not None:
        new_rows = jnp.concatenate([p1_rows, interp], axis=2)
    else:
        new_rows = interp
    cin = new_rows.shape[2]
    y, s = _layer([new_rows.reshape(rt, cin)], [layers[0]['W'].T], [None])
    mu, sig = _stats(y, s, rt)
    for lp in layers[1:]:
        y, s = _layer([y], [lp['W'].T], [(mu, sig)])
        mu, sig = _stats(y, s, rt)
    o = y.shape[1]
    out = _final(y, mu, sig, k=None)
    return out.reshape(b, n1, o)


def _forward(xyz, points, params, cfg):
    b = xyz.shape[0]
    xyz_rows = xyz.transpose(0, 2, 1)
    feat_rows = jnp.concatenate([xyz_rows, points.transpose(0, 2, 1)],
                                axis=2)
    sa_cfgs, fp_nts = cfg
    lx = [xyz_rows]
    lp = [None]
    cur_xyz_cn = xyz
    cur_xyz_rows = xyz_rows
    cur_feat = feat_rows
    for (name, npoint, radius, k, s_t) in sa_cfgs:
        new_xyz, pooled = _sa_stage(cur_xyz_cn, cur_xyz_rows, cur_feat,
                                    npoint, radius, k, params[name], s_t)
        lx.append(new_xyz)
        lp.append(pooled)
        cur_xyz_rows = new_xyz
        cur_xyz_cn = new_xyz.transpose(0, 2, 1)
        cur_feat = jnp.concatenate([new_xyz, pooled], axis=2)

    fp_names = ['fp4', 'fp3', 'fp2', 'fp1']
    feats = lp[4]
    for i, (name, n_t) in enumerate(zip(fp_names, fp_nts)):
        lvl = 3 - i
        p1 = lp[lvl] if lvl > 0 else None
        feats = _fp_stage(lx[lvl], lx[lvl + 1].transpose(0, 2, 1),
                          p1, feats, params[name], n_t)
    return feats.transpose(0, 2, 1)


_DEFAULT_CFG = (
    [('sa1', 4096, 1.0, 32, 256),
     ('sa2', 1024, 2.0, 32, 256),
     ('sa3', 256, 4.0, 32, 256),
     ('sa4', 64, 8.0, 32, 64)],
    [256, 1024, 2048, 512],
)


def kernel(xyz, points, params):
    return _forward(xyz, points, params, _DEFAULT_CFG)
